# baseline (speedup 1.0000x reference)
import functools

import jax
import jax.numpy as jnp
from jax import lax
from jax.experimental import pallas as pl
from jax.experimental.pallas import tpu as pltpu
from jax.experimental.pallas import tpu_sc as plsc

B = 16384
D = 32
NC = 2
NS = 16
NW = NC * NS
BPW = B // NW
L = 16


def _sc_body(user_hbm, movie_hbm, uf_hbm, mft_hbm, out_hbm,
             uidx, midx, urows, mvalsT, outv, su, sm):
    wid = lax.axis_index("s") * NC + lax.axis_index("c")
    base = wid * BPW
    pltpu.sync_copy(user_hbm.at[pl.ds(base, BPW)], uidx)
    pltpu.sync_copy(movie_hbm.at[pl.ds(base, BPW)], midx)

    cu = pltpu.async_copy(uf_hbm.at[uidx], urows, su)
    mcopies = []
    for f in range(D):
        mcopies.append(
            pltpu.async_copy(mft_hbm.at[f].at[midx], mvalsT.at[f], sm))
    cu.wait()
    for c in mcopies:
        c.wait()

    lane = lax.iota(jnp.int32, L)

    @pl.loop(0, BPW, step=L)
    def _(i):
        rows = lane + i
        acc = jnp.zeros((L,), jnp.float32)
        for f in range(D):
            uvals = plsc.load_gather(urows, [rows, jnp.full((L,), f, jnp.int32)])
            acc = acc + uvals * mvalsT[f, pl.ds(i, L)]
        outv[pl.ds(i, L)] = acc

    pltpu.sync_copy(outv, out_hbm.at[pl.ds(base, BPW)])


def kernel(user, movie, user_factors, movie_factors):
    mesh = plsc.VectorSubcoreMesh(core_axis_name="c", subcore_axis_name="s")
    kern = pl.kernel(
        _sc_body,
        out_type=jax.ShapeDtypeStruct((B,), jnp.float32),
        mesh=mesh,
        compiler_params=pltpu.CompilerParams(use_tc_tiling_on_sc=False,
                                             needs_layout_passes=False),
        scratch_types=[
            pltpu.VMEM((BPW,), jnp.int32),
            pltpu.VMEM((BPW,), jnp.int32),
            pltpu.VMEM((BPW, D), jnp.float32),
            pltpu.VMEM((D, BPW), jnp.float32),
            pltpu.VMEM((BPW,), jnp.float32),
            pltpu.SemaphoreType.DMA,
            pltpu.SemaphoreType.DMA,
        ],
    )
    return kern(user.astype(jnp.int32), movie.astype(jnp.int32),
                user_factors, movie_factors.T)
